# baseline (speedup 1.0000x reference)
import functools

import jax
import jax.numpy as jnp
import numpy as np
from jax.experimental import pallas as pl
from jax.experimental.pallas import tpu as pltpu


def _pool_conv_kernel(x_ref, p_ref, w_ref, o_ref):
    x = x_ref[0].astype(jnp.bfloat16)
    pooled = jnp.dot(x, p_ref[...], preferred_element_type=jnp.float32)
    out = jnp.dot(w_ref[...], pooled.astype(jnp.bfloat16),
                  preferred_element_type=jnp.float32)
    o_ref[0] = out.astype(o_ref.dtype)


@functools.lru_cache(maxsize=32)
def _pool_operator(th, scale, w_in, w_out):
    L, S = th * scale * w_in, th * w_out
    li, si = np.arange(L), np.arange(S)
    row_hit = (li[:, None] // (scale * w_in)) == (si[None, :] // w_out)
    col_hit = (li[:, None] % w_in) // scale == (si[None, :] % w_out)
    return ((row_hit & col_hit).astype(np.float32) / (scale * scale)).astype(
        jnp.bfloat16)


def _pick_th(hs, ws, w_in, scale, c, itemsize):
    vmem_budget = 24 * 2**20
    best = None
    for th in range(1, hs + 1):
        if hs % th:
            continue
        L, S = th * scale * w_in, th * ws
        if th != hs and (L % 128 or S % 128):
            continue
        need = 2 * itemsize * c * L + 2 * c * L + 2 * L * S
        if need <= vmem_budget:
            return th
        if best is None:
            best = th
    return best if best is not None else hs


def _run_pool_conv(x, w2d, *, scale):
    N, C, H, W = x.shape
    Hs, Ws = H // scale, W // scale
    C_out = w2d.shape[0]
    th = _pick_th(Hs, Ws, W, scale, C, x.dtype.itemsize)
    L, S = th * scale * W, th * Ws
    p_mat = jnp.asarray(_pool_operator(th, scale, W, Ws))
    x_flat = x.reshape(N, C, H * W)

    grid = (N, Hs // th)
    flops = grid[0] * grid[1] * 2 * (C * L * S + C_out * C * S)
    bytes_accessed = (x_flat.size * x_flat.dtype.itemsize
                      + N * C_out * Hs * Ws * x_flat.dtype.itemsize
                      + p_mat.size * 2 + w2d.size * 2)

    out_flat = pl.pallas_call(
        _pool_conv_kernel,
        out_shape=jax.ShapeDtypeStruct((N, C_out, Hs * Ws), x.dtype),
        grid=grid,
        in_specs=[
            pl.BlockSpec((1, C, L), lambda n, h: (n, 0, h)),
            pl.BlockSpec((L, S), lambda n, h: (0, 0)),
            pl.BlockSpec((C_out, C), lambda n, h: (0, 0)),
        ],
        out_specs=pl.BlockSpec((1, C_out, S), lambda n, h: (n, 0, h)),
        compiler_params=pltpu.CompilerParams(
            dimension_semantics=("parallel", "parallel"),
            vmem_limit_bytes=48 * 2**20,
        ),
        cost_estimate=pl.CostEstimate(flops=int(flops), transcendentals=0,
                                      bytes_accessed=int(bytes_accessed)),
    )(x_flat, p_mat, w2d)
    return out_flat.reshape(N, C_out, Hs, Ws)


def kernel(hidden_states, weight, *, scale=8):
    five_d = hidden_states.ndim == 5
    if five_d:
        B, F, C, H, W = hidden_states.shape
        x = hidden_states.reshape(B * F, C, H, W)
    else:
        x = hidden_states
    C_out, C_in = weight.shape[0], weight.shape[1]
    w2d = weight.reshape(C_out, C_in).astype(jnp.bfloat16)
    out = _run_pool_conv(x, w2d, scale=scale)
    if five_d:
        out = out.reshape(B, F, C_out, out.shape[-2], out.shape[-1])
    return out


# speedup vs baseline: 1.0479x; 1.0479x over previous
import functools

import jax
import jax.numpy as jnp
import numpy as np
from jax.experimental import pallas as pl
from jax.experimental.pallas import tpu as pltpu


def _pool_conv_kernel(x_ref, p_ref, w_ref, o_ref):
    x = x_ref[0].astype(jnp.bfloat16)
    pooled = jnp.dot(x, p_ref[...], preferred_element_type=jnp.float32)
    out = jnp.dot(w_ref[...], pooled.astype(jnp.bfloat16),
                  preferred_element_type=jnp.float32)
    o_ref[0] = out.astype(o_ref.dtype)


@functools.lru_cache(maxsize=32)
def _pool_operator(th, scale, w_in, w_out):
    L, S = th * scale * w_in, th * w_out
    li, si = np.arange(L), np.arange(S)
    row_hit = (li[:, None] // (scale * w_in)) == (si[None, :] // w_out)
    col_hit = (li[:, None] % w_in) // scale == (si[None, :] % w_out)
    return ((row_hit & col_hit).astype(np.float32) / (scale * scale)).astype(
        jnp.bfloat16)


def _pick_th(hs, ws, w_in, scale, c, itemsize):
    vmem_budget = 36 * 2**20
    best = None
    for th in range(1, hs + 1):
        if hs % th:
            continue
        L, S = th * scale * w_in, th * ws
        if th != hs and (L % 128 or S % 128):
            continue
        need = 2 * itemsize * c * L + 2 * c * L + 2 * L * S
        if need <= vmem_budget or best is None:
            best = th
        if need > vmem_budget and best is not None:
            break
    return best


def _run_pool_conv(x, w2d, *, scale):
    N, C, H, W = x.shape
    Hs, Ws = H // scale, W // scale
    C_out = w2d.shape[0]
    th = _pick_th(Hs, Ws, W, scale, C, x.dtype.itemsize)
    L, S = th * scale * W, th * Ws
    p_mat = jnp.asarray(_pool_operator(th, scale, W, Ws))
    x_flat = x.reshape(N, C, H * W)

    grid = (N, Hs // th)
    flops = grid[0] * grid[1] * 2 * (C * L * S + C_out * C * S)
    bytes_accessed = (x_flat.size * x_flat.dtype.itemsize
                      + N * C_out * Hs * Ws * x_flat.dtype.itemsize
                      + p_mat.size * 2 + w2d.size * 2)

    out_flat = pl.pallas_call(
        _pool_conv_kernel,
        out_shape=jax.ShapeDtypeStruct((N, C_out, Hs * Ws), x.dtype),
        grid=grid,
        in_specs=[
            pl.BlockSpec((1, C, L), lambda n, h: (n, 0, h)),
            pl.BlockSpec((L, S), lambda n, h: (0, 0)),
            pl.BlockSpec((C_out, C), lambda n, h: (0, 0)),
        ],
        out_specs=pl.BlockSpec((1, C_out, S), lambda n, h: (n, 0, h)),
        compiler_params=pltpu.CompilerParams(
            dimension_semantics=("parallel", "parallel"),
            vmem_limit_bytes=56 * 2**20,
        ),
        cost_estimate=pl.CostEstimate(flops=int(flops), transcendentals=0,
                                      bytes_accessed=int(bytes_accessed)),
    )(x_flat, p_mat, w2d)
    return out_flat.reshape(N, C_out, Hs, Ws)


def kernel(hidden_states, weight, *, scale=8):
    five_d = hidden_states.ndim == 5
    if five_d:
        B, F, C, H, W = hidden_states.shape
        x = hidden_states.reshape(B * F, C, H, W)
    else:
        x = hidden_states
    C_out, C_in = weight.shape[0], weight.shape[1]
    w2d = weight.reshape(C_out, C_in).astype(jnp.bfloat16)
    out = _run_pool_conv(x, w2d, scale=scale)
    if five_d:
        out = out.reshape(B, F, C_out, out.shape[-2], out.shape[-1])
    return out
